# baseline (speedup 1.0000x reference)
import jax
import jax.numpy as jnp
from jax import lax
from jax.experimental import pallas as pl
from jax.experimental.pallas import tpu as pltpu
from jax.experimental.pallas import tpu_sc as plsc

N = 10000
E = 320000
D = 128
H = 128
NQ = 16
NG = 8
NP = 6

NC = 2
NS = 16
NW = NC * NS
EPW = E // NW
CHUNK = 128
NCH = 80
SLOTS = NCH * CHUNK
NPAD = 10112
RPT = NPAD // NS
NPAD_D = 10240
RPT_D = NPAD_D // NS


def _deg_body(dstw, out, dstv, onesv, zv, cnt):
    c = lax.axis_index("c")
    s = lax.axis_index("s")
    wid = s * NC + c
    pltpu.sync_copy(dstw.at[wid], dstv)
    for i in range(CHUNK // 16):
        onesv[pl.ds(i * 16, 16)] = jnp.ones((16,), jnp.float32)
    for i in range(RPT_D // 16):
        zv[pl.ds(i * 16, 16)] = jnp.zeros((16,), jnp.float32)
    pltpu.sync_copy(zv, cnt.at[pl.ds(s * RPT_D, RPT_D)])
    plsc.subcore_barrier()

    def body(j, carry):
        pltpu.sync_copy(onesv, cnt.at[dstv.at[j]], add=True)
        return carry

    lax.fori_loop(0, NCH, body, 0)
    plsc.subcore_barrier()
    pltpu.sync_copy(cnt.at[pl.ds(s * RPT_D, RPT_D)],
                    out.at[c, pl.ds(s * RPT_D, RPT_D)])


def _scat_body(hs, ew, out, srcv, dstv, rows, acc, isem, gsem, ssem):
    c = lax.axis_index("c")
    s = lax.axis_index("s")
    wid = s * NC + c
    @pl.when(c == 0)
    def _():
        pltpu.sync_copy(hs.at[pl.ds(s * RPT, RPT)], acc.at[pl.ds(s * RPT, RPT)])

    @pl.when(c != 0)
    def _():
        def zrow(r, carry):
            for i in range(H // 16):
                rows[0, r, pl.ds(i * 16, 16)] = jnp.zeros((16,), jnp.float32)
            return carry

        lax.fori_loop(0, CHUNK, zrow, 0)
        for k in range(RPT // CHUNK):
            pltpu.sync_copy(
                rows.at[0], acc.at[pl.ds(s * RPT + k * CHUNK, CHUNK)])
        rem = RPT - (RPT // CHUNK) * CHUNK
        if rem:
            pltpu.sync_copy(
                rows.at[0, pl.ds(0, rem)],
                acc.at[pl.ds(s * RPT + (RPT // CHUNK) * CHUNK, rem)])

    def _fetch_idx(j):
        pltpu.async_copy(ew.at[wid, 0, j], srcv.at[j % 3], isem)
        pltpu.async_copy(ew.at[wid, 1, j], dstv.at[j % 4], isem)

    _fetch_idx(0)
    _fetch_idx(1)
    _fetch_idx(2)
    plsc.subcore_barrier()

    def _wait_idx():
        pltpu.make_async_copy(ew.at[wid, 0, 0], srcv.at[0], isem).wait()
        pltpu.make_async_copy(ew.at[wid, 0, 0], srcv.at[0], isem).wait()

    def _wait_gather():
        pltpu.make_async_copy(hs.at[pl.ds(0, CHUNK)], rows.at[0], gsem).wait()

    def _wait_scatter():
        pltpu.make_async_copy(hs.at[pl.ds(0, CHUNK)], rows.at[0], ssem).wait()

    _wait_idx()
    pltpu.async_copy(hs.at[srcv.at[0]], rows.at[0], gsem)
    _wait_idx()
    pltpu.async_copy(hs.at[srcv.at[1]], rows.at[1], gsem)

    def body(j, carry):
        b = j % 3
        _wait_gather()

        @pl.when(j > 0)
        def _():
            _wait_scatter()

        pltpu.async_copy(rows.at[b], acc.at[dstv.at[j % 4]], ssem, add=True)

        @pl.when(j + 2 < NCH)
        def _():
            _wait_idx()
            pltpu.async_copy(hs.at[srcv.at[(j + 2) % 3]],
                             rows.at[(j + 2) % 3], gsem)

        @pl.when(j + 3 < NCH)
        def _():
            _fetch_idx(j + 3)

        return carry

    lax.fori_loop(0, NCH, body, 0)
    _wait_scatter()
    plsc.subcore_barrier()
    pltpu.sync_copy(acc.at[pl.ds(s * RPT, RPT)], out.at[c, pl.ds(s * RPT, RPT)])


_sc_mesh = plsc.VectorSubcoreMesh(core_axis_name="c", subcore_axis_name="s")


def _deg(dstw):
    return pl.kernel(
        _deg_body,
        out_type=jax.ShapeDtypeStruct((NC, NPAD_D), jnp.float32),
        mesh=_sc_mesh,
        scratch_types=[
            pltpu.VMEM((NCH, CHUNK), jnp.int32),
            pltpu.VMEM((CHUNK,), jnp.float32),
            pltpu.VMEM((RPT_D,), jnp.float32),
            pltpu.VMEM_SHARED((NPAD_D,), jnp.float32),
        ],
    )(dstw)


def _scatter(hs, ew):
    return pl.kernel(
        _scat_body,
        out_type=jax.ShapeDtypeStruct((NC, NPAD, H), jnp.float32),
        mesh=_sc_mesh,
        scratch_types=[
            pltpu.VMEM((3, CHUNK), jnp.int32),
            pltpu.VMEM((4, CHUNK), jnp.int32),
            pltpu.VMEM((3, CHUNK, H), jnp.float32),
            pltpu.VMEM_SHARED((NPAD, H), jnp.float32),
            pltpu.SemaphoreType.DMA,
            pltpu.SemaphoreType.DMA,
            pltpu.SemaphoreType.DMA,
        ],
    )(hs, ew)


RB = 128
G = NPAD // RB


def _tc1_body(x_ref, w1_ref, cnt_ref, hs_ref, dinv_ref):
    dinv = lax.rsqrt(cnt_ref[0] + cnt_ref[1] + 1.0)[:, None]
    h = jnp.dot(x_ref[...], w1_ref[...], preferred_element_type=jnp.float32)
    hs_ref[...] = h * dinv
    dinv_ref[...] = dinv


def _tc2_body(a_ref, dinv_ref, b1_ref, w2_ref, hs2_ref):
    agg = a_ref[0] + a_ref[1]
    dinv = dinv_ref[...]
    h = jnp.maximum(agg * dinv + b1_ref[...], 0.0)
    hs2_ref[...] = jnp.dot(h, w2_ref[...], preferred_element_type=jnp.float32) * dinv


def _tc3_body(a_ref, dinv_ref, b2_ref, wl_ref, bl_ref, wq_ref, bq_ref,
              wg_ref, bg_ref, wp_ref, bp_ref, wt_ref, bt_ref,
              q_ref, g_ref, p_ref, t_ref, vsum):
    i = pl.program_id(0)

    @pl.when(i == 0)
    def _():
        vsum[...] = jnp.zeros((1, H), jnp.float32)

    agg = a_ref[0] + a_ref[1]
    h2 = jnp.maximum(agg * dinv_ref[...] + b2_ref[...], 0.0)
    rowid = i * RB + lax.broadcasted_iota(jnp.int32, (RB, 1), 0)
    h2 = jnp.where(rowid < N, h2, 0.0)
    vsum[...] += jnp.sum(h2, axis=0, keepdims=True)

    @pl.when(i == G - 1)
    def _():
        v = vsum[...] * (1.0 / N)
        u = jnp.maximum(
            jnp.dot(v, wl_ref[...], preferred_element_type=jnp.float32)
            + bl_ref[...], 0.0)
        q_ref[...] = jnp.dot(u, wq_ref[...], preferred_element_type=jnp.float32) + bq_ref[...]
        g_ref[...] = jnp.dot(u, wg_ref[...], preferred_element_type=jnp.float32) + bg_ref[...]
        p_ref[...] = jnp.dot(u, wp_ref[...], preferred_element_type=jnp.float32) + bp_ref[...]
        t_ref[...] = jnp.dot(u, wt_ref[...], preferred_element_type=jnp.float32) + bt_ref[...]


def _f32(shape):
    return jax.ShapeDtypeStruct(shape, jnp.float32)


def kernel(x, edge_index, W1, b1, W2, b2, Wl, bl, Wq, bq, Wg, bg, Wp, bp, Wt, bt):
    src = edge_index[0].reshape(NW, EPW)
    dst = edge_index[1].reshape(NW, EPW)
    npads = NW * (SLOTS - EPW)
    pad_src = (jnp.arange(npads, dtype=jnp.int32) % N).reshape(NW, SLOTS - EPW)
    pad_dst = (N + (jnp.arange(npads, dtype=jnp.int32) % (NPAD - N))).reshape(
        NW, SLOTS - EPW)
    srcw = jnp.concatenate([src, pad_src], axis=1).reshape(NW, NCH, CHUNK)
    dstw = jnp.concatenate([dst, pad_dst], axis=1).reshape(NW, NCH, CHUNK)
    ew = jnp.stack([srcw, dstw], axis=1)
    x_p = jnp.pad(x, ((0, NPAD - N), (0, 0)))

    cnt = _deg(dstw)

    hs1, dinv = pl.pallas_call(
        _tc1_body,
        grid=(G,),
        in_specs=[
            pl.BlockSpec((RB, D), lambda i: (i, 0)),
            pl.BlockSpec((D, H), lambda i: (0, 0)),
            pl.BlockSpec((NC, RB), lambda i: (0, i)),
        ],
        out_specs=(
            pl.BlockSpec((RB, H), lambda i: (i, 0)),
            pl.BlockSpec((RB, 1), lambda i: (i, 0)),
        ),
        out_shape=(_f32((NPAD, H)), _f32((NPAD, 1))),
    )(x_p, W1, cnt)

    acc1 = _scatter(hs1, ew)

    hs2 = pl.pallas_call(
        _tc2_body,
        grid=(G,),
        in_specs=[
            pl.BlockSpec((NC, RB, H), lambda i: (0, i, 0)),
            pl.BlockSpec((RB, 1), lambda i: (i, 0)),
            pl.BlockSpec((1, H), lambda i: (0, 0)),
            pl.BlockSpec((H, H), lambda i: (0, 0)),
        ],
        out_specs=pl.BlockSpec((RB, H), lambda i: (i, 0)),
        out_shape=_f32((NPAD, H)),
    )(acc1, dinv, b1.reshape(1, H), W2)

    acc2 = _scatter(hs2, ew)

    _fix = lambda i: (0, 0)
    q, g, p, t = pl.pallas_call(
        _tc3_body,
        grid=(G,),
        in_specs=[
            pl.BlockSpec((NC, RB, H), lambda i: (0, i, 0)),
            pl.BlockSpec((RB, 1), lambda i: (i, 0)),
            pl.BlockSpec((1, H), _fix),
            pl.BlockSpec((H, H), _fix),
            pl.BlockSpec((1, H), _fix),
            pl.BlockSpec((H, NQ), _fix),
            pl.BlockSpec((1, NQ), _fix),
            pl.BlockSpec((H, NG), _fix),
            pl.BlockSpec((1, NG), _fix),
            pl.BlockSpec((H, NP), _fix),
            pl.BlockSpec((1, NP), _fix),
            pl.BlockSpec((H, NQ - 1), _fix),
            pl.BlockSpec((1, NQ - 1), _fix),
        ],
        out_specs=(
            pl.BlockSpec((1, NQ), _fix),
            pl.BlockSpec((1, NG), _fix),
            pl.BlockSpec((1, NP), _fix),
            pl.BlockSpec((1, NQ - 1), _fix),
        ),
        out_shape=(_f32((1, NQ)), _f32((1, NG)), _f32((1, NP)), _f32((1, NQ - 1))),
        scratch_shapes=[pltpu.VMEM((1, H), jnp.float32)],
    )(acc2, dinv, b2.reshape(1, H), Wl, bl.reshape(1, H),
      Wq, bq.reshape(1, NQ), Wg, bg.reshape(1, NG),
      Wp, bp.reshape(1, NP), Wt, bt.reshape(1, NQ - 1))
    return (q.reshape(NQ), g.reshape(NG), p.reshape(NP), t.reshape(NQ - 1))


# speedup vs baseline: 1.3797x; 1.3797x over previous
import jax
import jax.numpy as jnp
from jax import lax
from jax.experimental import pallas as pl
from jax.experimental.pallas import tpu as pltpu
from jax.experimental.pallas import tpu_sc as plsc

N = 10000
E = 320000
D = 128
H = 128
NQ = 16
NG = 8
NP = 6

NC = 2
NS = 16
NW = NC * NS
EPW = E // NW
CHUNK = 128
NCH = 80
SLOTS = NCH * CHUNK
NPAD = 10112
RPT = NPAD // NS
NPAD_D = 10240
RPT_D = NPAD_D // NS


def _deg_body(dstw, out, dstv, onesv, zv, cnt):
    c = lax.axis_index("c")
    s = lax.axis_index("s")
    wid = s * NC + c
    pltpu.sync_copy(dstw.at[wid], dstv)
    for i in range(CHUNK // 16):
        onesv[pl.ds(i * 16, 16)] = jnp.ones((16,), jnp.float32)
    for i in range(RPT_D // 16):
        zv[pl.ds(i * 16, 16)] = jnp.zeros((16,), jnp.float32)
    pltpu.sync_copy(zv, cnt.at[pl.ds(s * RPT_D, RPT_D)])
    plsc.subcore_barrier()

    def body(j, carry):
        pltpu.sync_copy(onesv, cnt.at[dstv.at[j]], add=True)
        return carry

    lax.fori_loop(0, NCH, body, 0)
    plsc.subcore_barrier()
    pltpu.sync_copy(cnt.at[pl.ds(s * RPT_D, RPT_D)],
                    out.at[c, pl.ds(s * RPT_D, RPT_D)])


def _scat_body(hs, ew, out, srcv, dstv, rows, acc, isem, gsem, ssem):
    c = lax.axis_index("c")
    s = lax.axis_index("s")
    wid = s * NC + c
    @pl.when(c == 0)
    def _():
        pltpu.sync_copy(hs.at[pl.ds(s * RPT, RPT)], acc.at[pl.ds(s * RPT, RPT)])

    @pl.when(c != 0)
    def _():
        def zrow(r, carry):
            for i in range(H // 16):
                rows[0, r, pl.ds(i * 16, 16)] = jnp.zeros((16,), jnp.float32)
            return carry

        lax.fori_loop(0, CHUNK, zrow, 0)
        for k in range(RPT // CHUNK):
            pltpu.sync_copy(
                rows.at[0], acc.at[pl.ds(s * RPT + k * CHUNK, CHUNK)])
        rem = RPT - (RPT // CHUNK) * CHUNK
        if rem:
            pltpu.sync_copy(
                rows.at[0, pl.ds(0, rem)],
                acc.at[pl.ds(s * RPT + (RPT // CHUNK) * CHUNK, rem)])

    def _fetch_idx(j):
        pltpu.async_copy(ew.at[wid, 0, j], srcv.at[j % 3], isem)
        pltpu.async_copy(ew.at[wid, 1, j], dstv.at[j % 4], isem)

    _fetch_idx(0)
    _fetch_idx(1)
    _fetch_idx(2)
    plsc.subcore_barrier()

    def _wait_idx():
        pltpu.make_async_copy(ew.at[wid, 0, 0], srcv.at[0], isem).wait()
        pltpu.make_async_copy(ew.at[wid, 0, 0], srcv.at[0], isem).wait()

    def _wait_gather():
        pltpu.make_async_copy(hs.at[pl.ds(0, CHUNK)], rows.at[0], gsem).wait()

    def _wait_scatter():
        pltpu.make_async_copy(hs.at[pl.ds(0, CHUNK)], rows.at[0], ssem).wait()

    _wait_idx()
    pltpu.async_copy(hs.at[srcv.at[0]], rows.at[0], gsem)
    _wait_idx()
    pltpu.async_copy(hs.at[srcv.at[1]], rows.at[1], gsem)

    def body(j, carry):
        b = j % 3
        _wait_gather()

        @pl.when(j > 0)
        def _():
            _wait_scatter()

        pltpu.async_copy(rows.at[b], acc.at[dstv.at[j % 4]], ssem, add=True)

        @pl.when(j + 2 < NCH)
        def _():
            _wait_idx()
            pltpu.async_copy(hs.at[srcv.at[(j + 2) % 3]],
                             rows.at[(j + 2) % 3], gsem)

        @pl.when(j + 3 < NCH)
        def _():
            _fetch_idx(j + 3)

        return carry

    lax.fori_loop(0, NCH, body, 0)
    _wait_scatter()
    plsc.subcore_barrier()
    pltpu.sync_copy(acc.at[pl.ds(s * RPT, RPT)], out.at[c, pl.ds(s * RPT, RPT)])


_sc_mesh = plsc.VectorSubcoreMesh(core_axis_name="c", subcore_axis_name="s")


def _deg(dstw):
    return pl.kernel(
        _deg_body,
        out_type=jax.ShapeDtypeStruct((NC, NPAD_D), jnp.float32),
        mesh=_sc_mesh,
        scratch_types=[
            pltpu.VMEM((NCH, CHUNK), jnp.int32),
            pltpu.VMEM((CHUNK,), jnp.float32),
            pltpu.VMEM((RPT_D,), jnp.float32),
            pltpu.VMEM_SHARED((NPAD_D,), jnp.float32),
        ],
    )(dstw)


def _scatter(hs, ew):
    return pl.kernel(
        _scat_body,
        out_type=jax.ShapeDtypeStruct((NC, NPAD, H), jnp.float32),
        mesh=_sc_mesh,
        scratch_types=[
            pltpu.VMEM((3, CHUNK), jnp.int32),
            pltpu.VMEM((4, CHUNK), jnp.int32),
            pltpu.VMEM((3, CHUNK, H), jnp.float32),
            pltpu.VMEM_SHARED((NPAD, H), jnp.float32),
            pltpu.SemaphoreType.DMA,
            pltpu.SemaphoreType.DMA,
            pltpu.SemaphoreType.DMA,
        ],
    )(hs, ew)


RB = 632
G = NPAD // RB


def _tc1_body(x_ref, w1_ref, cnt_ref, hs_ref, dinv_ref):
    cnt = cnt_ref[0, :NPAD] + cnt_ref[1, :NPAD]
    dinv = lax.rsqrt(cnt + 1.0)[:, None]
    h = jnp.dot(x_ref[...], w1_ref[...], preferred_element_type=jnp.float32)
    hs_ref[...] = h * dinv
    dinv_ref[...] = dinv


def _tc2_body(a_ref, dinv_ref, b1_ref, w2_ref, hs2_ref):
    agg = a_ref[0] + a_ref[1]
    dinv = dinv_ref[...]
    h = jnp.maximum(agg * dinv + b1_ref[...], 0.0)
    hs2_ref[...] = jnp.dot(h, w2_ref[...], preferred_element_type=jnp.float32) * dinv


def _tc3_body(a_ref, dinv_ref, b2_ref, wl_ref, bl_ref, wq_ref, bq_ref,
              wg_ref, bg_ref, wp_ref, bp_ref, wt_ref, bt_ref,
              q_ref, g_ref, p_ref, t_ref, vsum):
    i = pl.program_id(0)

    @pl.when(i == 0)
    def _():
        vsum[...] = jnp.zeros((1, H), jnp.float32)

    agg = a_ref[0] + a_ref[1]
    h2 = jnp.maximum(agg * dinv_ref[...] + b2_ref[...], 0.0)
    rowid = i * RB + lax.broadcasted_iota(jnp.int32, (RB, 1), 0)
    h2 = jnp.where(rowid < N, h2, 0.0)
    vsum[...] += jnp.sum(h2, axis=0, keepdims=True)

    @pl.when(i == G - 1)
    def _():
        v = vsum[...] * (1.0 / N)
        u = jnp.maximum(
            jnp.dot(v, wl_ref[...], preferred_element_type=jnp.float32)
            + bl_ref[...], 0.0)
        q_ref[...] = jnp.dot(u, wq_ref[...], preferred_element_type=jnp.float32) + bq_ref[...]
        g_ref[...] = jnp.dot(u, wg_ref[...], preferred_element_type=jnp.float32) + bg_ref[...]
        p_ref[...] = jnp.dot(u, wp_ref[...], preferred_element_type=jnp.float32) + bp_ref[...]
        t_ref[...] = jnp.dot(u, wt_ref[...], preferred_element_type=jnp.float32) + bt_ref[...]


def _f32(shape):
    return jax.ShapeDtypeStruct(shape, jnp.float32)


def kernel(x, edge_index, W1, b1, W2, b2, Wl, bl, Wq, bq, Wg, bg, Wp, bp, Wt, bt):
    src = edge_index[0].reshape(NW, EPW)
    dst = edge_index[1].reshape(NW, EPW)
    npads = NW * (SLOTS - EPW)
    pad_src = (jnp.arange(npads, dtype=jnp.int32) % N).reshape(NW, SLOTS - EPW)
    pad_dst = (N + (jnp.arange(npads, dtype=jnp.int32) % (NPAD - N))).reshape(
        NW, SLOTS - EPW)
    srcw = jnp.concatenate([src, pad_src], axis=1).reshape(NW, NCH, CHUNK)
    dstw = jnp.concatenate([dst, pad_dst], axis=1).reshape(NW, NCH, CHUNK)
    ew = jnp.stack([srcw, dstw], axis=1)
    x_p = jnp.pad(x, ((0, NPAD - N), (0, 0)))

    cnt = _deg(dstw)

    hs1, dinv = pl.pallas_call(
        _tc1_body,
        out_shape=(_f32((NPAD, H)), _f32((NPAD, 1))),
    )(x_p, W1, cnt)

    acc1 = _scatter(hs1, ew)

    hs2 = pl.pallas_call(
        _tc2_body,
        grid=(G,),
        in_specs=[
            pl.BlockSpec((NC, RB, H), lambda i: (0, i, 0)),
            pl.BlockSpec((RB, 1), lambda i: (i, 0)),
            pl.BlockSpec((1, H), lambda i: (0, 0)),
            pl.BlockSpec((H, H), lambda i: (0, 0)),
        ],
        out_specs=pl.BlockSpec((RB, H), lambda i: (i, 0)),
        out_shape=_f32((NPAD, H)),
    )(acc1, dinv, b1.reshape(1, H), W2)

    acc2 = _scatter(hs2, ew)

    _fix = lambda i: (0, 0)
    q, g, p, t = pl.pallas_call(
        _tc3_body,
        grid=(G,),
        in_specs=[
            pl.BlockSpec((NC, RB, H), lambda i: (0, i, 0)),
            pl.BlockSpec((RB, 1), lambda i: (i, 0)),
            pl.BlockSpec((1, H), _fix),
            pl.BlockSpec((H, H), _fix),
            pl.BlockSpec((1, H), _fix),
            pl.BlockSpec((H, NQ), _fix),
            pl.BlockSpec((1, NQ), _fix),
            pl.BlockSpec((H, NG), _fix),
            pl.BlockSpec((1, NG), _fix),
            pl.BlockSpec((H, NP), _fix),
            pl.BlockSpec((1, NP), _fix),
            pl.BlockSpec((H, NQ - 1), _fix),
            pl.BlockSpec((1, NQ - 1), _fix),
        ],
        out_specs=(
            pl.BlockSpec((1, NQ), _fix),
            pl.BlockSpec((1, NG), _fix),
            pl.BlockSpec((1, NP), _fix),
            pl.BlockSpec((1, NQ - 1), _fix),
        ),
        out_shape=(_f32((1, NQ)), _f32((1, NG)), _f32((1, NP)), _f32((1, NQ - 1))),
        scratch_shapes=[pltpu.VMEM((1, H), jnp.float32)],
    )(acc2, dinv, b2.reshape(1, H), Wl, bl.reshape(1, H),
      Wq, bq.reshape(1, NQ), Wg, bg.reshape(1, NG),
      Wp, bp.reshape(1, NP), Wt, bt.reshape(1, NQ - 1))
    return (q.reshape(NQ), g.reshape(NG), p.reshape(NP), t.reshape(NQ - 1))


# speedup vs baseline: 1.4612x; 1.0591x over previous
import jax
import jax.numpy as jnp
from jax import lax
from jax.experimental import pallas as pl
from jax.experimental.pallas import tpu as pltpu
from jax.experimental.pallas import tpu_sc as plsc

N = 10000
E = 320000
D = 128
H = 128
NQ = 16
NG = 8
NP = 6

NC = 2
NS = 16
NW = NC * NS
EPW = E // NW
CHUNK = 128
NCH = 80
SLOTS = NCH * CHUNK
NPAD = 10112
RPT = NPAD // NS
NPAD_D = 10240
RPT_D = NPAD_D // NS


def _deg_body(dstw, out, dstv, onesv, zv, cnt, dsem):
    c = lax.axis_index("c")
    s = lax.axis_index("s")
    wid = s * NC + c
    pltpu.sync_copy(dstw.at[wid], dstv)
    for i in range(CHUNK // 16):
        onesv[pl.ds(i * 16, 16)] = jnp.ones((16,), jnp.float32)
    for i in range(RPT_D // 16):
        zv[pl.ds(i * 16, 16)] = jnp.zeros((16,), jnp.float32)
    pltpu.sync_copy(zv, cnt.at[pl.ds(s * RPT_D, RPT_D)])
    plsc.subcore_barrier()

    def body(j, carry):
        pltpu.async_copy(onesv, cnt.at[dstv.at[j]], dsem, add=True)

        @pl.when(j >= 8)
        def _():
            pltpu.make_async_copy(onesv, cnt.at[dstv.at[0]], dsem).wait()

        return carry

    lax.fori_loop(0, NCH, body, 0)
    for _ in range(8):
        pltpu.make_async_copy(onesv, cnt.at[dstv.at[0]], dsem).wait()
    plsc.subcore_barrier()
    pltpu.sync_copy(cnt.at[pl.ds(s * RPT_D, RPT_D)],
                    out.at[c, pl.ds(s * RPT_D, RPT_D)])


def _scat_body(hs, ew, out, srcv, dstv, rows, acc, isem, gsem, ssem):
    c = lax.axis_index("c")
    s = lax.axis_index("s")
    wid = s * NC + c
    @pl.when(c == 0)
    def _():
        pltpu.sync_copy(hs.at[pl.ds(s * RPT, RPT)], acc.at[pl.ds(s * RPT, RPT)])

    @pl.when(c != 0)
    def _():
        def zrow(r, carry):
            for i in range(H // 16):
                rows[0, r, pl.ds(i * 16, 16)] = jnp.zeros((16,), jnp.float32)
            return carry

        lax.fori_loop(0, CHUNK, zrow, 0)
        for k in range(RPT // CHUNK):
            pltpu.sync_copy(
                rows.at[0], acc.at[pl.ds(s * RPT + k * CHUNK, CHUNK)])
        rem = RPT - (RPT // CHUNK) * CHUNK
        if rem:
            pltpu.sync_copy(
                rows.at[0, pl.ds(0, rem)],
                acc.at[pl.ds(s * RPT + (RPT // CHUNK) * CHUNK, rem)])

    def _fetch_idx(j):
        pltpu.async_copy(ew.at[wid, 0, j], srcv.at[j % 3], isem)
        pltpu.async_copy(ew.at[wid, 1, j], dstv.at[j % 4], isem)

    _fetch_idx(0)
    _fetch_idx(1)
    _fetch_idx(2)
    plsc.subcore_barrier()

    def _wait_idx():
        pltpu.make_async_copy(ew.at[wid, 0, 0], srcv.at[0], isem).wait()
        pltpu.make_async_copy(ew.at[wid, 0, 0], srcv.at[0], isem).wait()

    def _wait_gather():
        pltpu.make_async_copy(hs.at[pl.ds(0, CHUNK)], rows.at[0], gsem).wait()

    def _wait_scatter():
        pltpu.make_async_copy(hs.at[pl.ds(0, CHUNK)], rows.at[0], ssem).wait()

    _wait_idx()
    pltpu.async_copy(hs.at[srcv.at[0]], rows.at[0], gsem)
    _wait_idx()
    pltpu.async_copy(hs.at[srcv.at[1]], rows.at[1], gsem)

    def body(j, carry):
        b = j % 3
        _wait_gather()

        @pl.when(j > 0)
        def _():
            _wait_scatter()

        pltpu.async_copy(rows.at[b], acc.at[dstv.at[j % 4]], ssem, add=True)

        @pl.when(j + 2 < NCH)
        def _():
            _wait_idx()
            pltpu.async_copy(hs.at[srcv.at[(j + 2) % 3]],
                             rows.at[(j + 2) % 3], gsem)

        @pl.when(j + 3 < NCH)
        def _():
            _fetch_idx(j + 3)

        return carry

    lax.fori_loop(0, NCH, body, 0)
    _wait_scatter()
    plsc.subcore_barrier()
    pltpu.sync_copy(acc.at[pl.ds(s * RPT, RPT)], out.at[c, pl.ds(s * RPT, RPT)])


_sc_mesh = plsc.VectorSubcoreMesh(core_axis_name="c", subcore_axis_name="s")


def _deg(dstw):
    return pl.kernel(
        _deg_body,
        out_type=jax.ShapeDtypeStruct((NC, NPAD_D), jnp.float32),
        mesh=_sc_mesh,
        scratch_types=[
            pltpu.VMEM((NCH, CHUNK), jnp.int32),
            pltpu.VMEM((CHUNK,), jnp.float32),
            pltpu.VMEM((RPT_D,), jnp.float32),
            pltpu.VMEM_SHARED((NPAD_D,), jnp.float32),
            pltpu.SemaphoreType.DMA,
        ],
    )(dstw)


def _scatter(hs, ew):
    return pl.kernel(
        _scat_body,
        out_type=jax.ShapeDtypeStruct((NC, NPAD, H), jnp.float32),
        mesh=_sc_mesh,
        scratch_types=[
            pltpu.VMEM((3, CHUNK), jnp.int32),
            pltpu.VMEM((4, CHUNK), jnp.int32),
            pltpu.VMEM((3, CHUNK, H), jnp.float32),
            pltpu.VMEM_SHARED((NPAD, H), jnp.float32),
            pltpu.SemaphoreType.DMA,
            pltpu.SemaphoreType.DMA,
            pltpu.SemaphoreType.DMA,
        ],
    )(hs, ew)


def _tc1_body(x_ref, w1_ref, cnt_ref, hs_ref, dinv_ref):
    cnt = cnt_ref[0, :NPAD] + cnt_ref[1, :NPAD]
    dinv = lax.rsqrt(cnt + 1.0)[:, None]
    h = jnp.dot(x_ref[...], w1_ref[...], preferred_element_type=jnp.float32)
    hs_ref[...] = h * dinv
    dinv_ref[...] = dinv


def _tc2_body(a_ref, dinv_ref, b1_ref, w2_ref, hs2_ref):
    agg = a_ref[0] + a_ref[1]
    dinv = dinv_ref[...]
    h = jnp.maximum(agg * dinv + b1_ref[...], 0.0)
    hs2_ref[...] = jnp.dot(h, w2_ref[...], preferred_element_type=jnp.float32) * dinv


def _tc3_body(a_ref, dinv_ref, b2_ref, wl_ref, bl_ref, wq_ref, bq_ref,
              wg_ref, bg_ref, wp_ref, bp_ref, wt_ref, bt_ref,
              q_ref, g_ref, p_ref, t_ref):
    agg = a_ref[0] + a_ref[1]
    h2 = jnp.maximum(agg * dinv_ref[...] + b2_ref[...], 0.0)
    v = jnp.sum(h2[:N, :], axis=0, keepdims=True) * (1.0 / N)
    u = jnp.maximum(
        jnp.dot(v, wl_ref[...], preferred_element_type=jnp.float32)
        + bl_ref[...], 0.0)
    q_ref[...] = jnp.dot(u, wq_ref[...], preferred_element_type=jnp.float32) + bq_ref[...]
    g_ref[...] = jnp.dot(u, wg_ref[...], preferred_element_type=jnp.float32) + bg_ref[...]
    p_ref[...] = jnp.dot(u, wp_ref[...], preferred_element_type=jnp.float32) + bp_ref[...]
    t_ref[...] = jnp.dot(u, wt_ref[...], preferred_element_type=jnp.float32) + bt_ref[...]


def _f32(shape):
    return jax.ShapeDtypeStruct(shape, jnp.float32)


def kernel(x, edge_index, W1, b1, W2, b2, Wl, bl, Wq, bq, Wg, bg, Wp, bp, Wt, bt):
    src = edge_index[0].reshape(NW, EPW)
    dst = edge_index[1].reshape(NW, EPW)
    npads = NW * (SLOTS - EPW)
    pad_src = (jnp.arange(npads, dtype=jnp.int32) % N).reshape(NW, SLOTS - EPW)
    pad_dst = (N + (jnp.arange(npads, dtype=jnp.int32) % (NPAD - N))).reshape(
        NW, SLOTS - EPW)
    srcw = jnp.concatenate([src, pad_src], axis=1).reshape(NW, NCH, CHUNK)
    dstw = jnp.concatenate([dst, pad_dst], axis=1).reshape(NW, NCH, CHUNK)
    ew = jnp.stack([srcw, dstw], axis=1)
    x_p = jnp.pad(x, ((0, NPAD - N), (0, 0)))

    cnt = _deg(dstw)

    hs1, dinv = pl.pallas_call(
        _tc1_body,
        out_shape=(_f32((NPAD, H)), _f32((NPAD, 1))),
    )(x_p, W1, cnt)

    acc1 = _scatter(hs1, ew)

    hs2 = pl.pallas_call(
        _tc2_body,
        out_shape=_f32((NPAD, H)),
    )(acc1, dinv, b1.reshape(1, H), W2)

    acc2 = _scatter(hs2, ew)

    q, g, p, t = pl.pallas_call(
        _tc3_body,
        out_shape=(_f32((1, NQ)), _f32((1, NG)), _f32((1, NP)), _f32((1, NQ - 1))),
    )(acc2, dinv, b2.reshape(1, H), Wl, bl.reshape(1, H),
      Wq, bq.reshape(1, NQ), Wg, bg.reshape(1, NG),
      Wp, bp.reshape(1, NP), Wt, bt.reshape(1, NQ - 1))
    return (q.reshape(NQ), g.reshape(NG), p.reshape(NP), t.reshape(NQ - 1))


# speedup vs baseline: 1.4666x; 1.0037x over previous
import jax
import jax.numpy as jnp
from jax import lax
from jax.experimental import pallas as pl
from jax.experimental.pallas import tpu as pltpu
from jax.experimental.pallas import tpu_sc as plsc

N = 10000
E = 320000
D = 128
H = 128
NQ = 16
NG = 8
NP = 6

NC = 2
NS = 16
NW = NC * NS
EPW = E // NW
CHUNK = 128
NCH = 80
SLOTS = NCH * CHUNK
NPAD = 10112
RPT = NPAD // NS
NPAD_D = 10240
RPT_D = NPAD_D // NS


def _deg_body(dstw, out, dstv, onesv, zv, cnt, dsem):
    c = lax.axis_index("c")
    s = lax.axis_index("s")
    wid = s * NC + c
    pltpu.sync_copy(dstw.at[wid], dstv)
    for i in range(CHUNK // 16):
        onesv[pl.ds(i * 16, 16)] = jnp.ones((16,), jnp.float32)
    for i in range(RPT_D // 16):
        zv[pl.ds(i * 16, 16)] = jnp.zeros((16,), jnp.float32)
    pltpu.sync_copy(zv, cnt.at[pl.ds(s * RPT_D, RPT_D)])
    plsc.subcore_barrier()

    def body(j, carry):
        pltpu.async_copy(onesv, cnt.at[dstv.at[j]], dsem, add=True)

        @pl.when(j >= 8)
        def _():
            pltpu.make_async_copy(onesv, cnt.at[dstv.at[0]], dsem).wait()

        return carry

    lax.fori_loop(0, NCH, body, 0)
    for _ in range(8):
        pltpu.make_async_copy(onesv, cnt.at[dstv.at[0]], dsem).wait()
    plsc.subcore_barrier()
    pltpu.sync_copy(cnt.at[pl.ds(s * RPT_D, RPT_D)],
                    out.at[c, pl.ds(s * RPT_D, RPT_D)])


def _scat_body(hs, ew, out, srcv, dstv, rows, acc, isem, gsem, ssem):
    c = lax.axis_index("c")
    s = lax.axis_index("s")
    wid = s * NC + c
    @pl.when(c == 0)
    def _():
        pltpu.sync_copy(hs.at[pl.ds(s * RPT, RPT)], acc.at[pl.ds(s * RPT, RPT)])

    @pl.when(c != 0)
    def _():
        def zrow(r, carry):
            for i in range(H // 16):
                rows[0, r, pl.ds(i * 16, 16)] = jnp.zeros((16,), jnp.float32)
            return carry

        lax.fori_loop(0, CHUNK, zrow, 0)
        for k in range(RPT // CHUNK):
            pltpu.sync_copy(
                rows.at[0], acc.at[pl.ds(s * RPT + k * CHUNK, CHUNK)])
        rem = RPT - (RPT // CHUNK) * CHUNK
        if rem:
            pltpu.sync_copy(
                rows.at[0, pl.ds(0, rem)],
                acc.at[pl.ds(s * RPT + (RPT // CHUNK) * CHUNK, rem)])

    def _fetch_idx(j):
        pltpu.async_copy(ew.at[wid, 0, j], srcv.at[j % 3], isem)
        pltpu.async_copy(ew.at[wid, 1, j], dstv.at[j % 4], isem)

    _fetch_idx(0)
    _fetch_idx(1)
    _fetch_idx(2)

    def _wait_idx():
        pltpu.make_async_copy(ew.at[wid, 0, 0], srcv.at[0], isem).wait()
        pltpu.make_async_copy(ew.at[wid, 0, 0], srcv.at[0], isem).wait()

    def _wait_gather():
        pltpu.make_async_copy(hs.at[pl.ds(0, CHUNK)], rows.at[0], gsem).wait()

    def _wait_scatter():
        pltpu.make_async_copy(hs.at[pl.ds(0, CHUNK)], rows.at[0], ssem).wait()

    _wait_idx()
    pltpu.async_copy(hs.at[srcv.at[0]], rows.at[0], gsem)
    _wait_idx()
    pltpu.async_copy(hs.at[srcv.at[1]], rows.at[1], gsem)
    plsc.subcore_barrier()

    def body(j, carry):
        b = j % 3
        _wait_gather()

        @pl.when(j > 0)
        def _():
            _wait_scatter()

        pltpu.async_copy(rows.at[b], acc.at[dstv.at[j % 4]], ssem, add=True)

        @pl.when(j + 2 < NCH)
        def _():
            _wait_idx()
            pltpu.async_copy(hs.at[srcv.at[(j + 2) % 3]],
                             rows.at[(j + 2) % 3], gsem)

        @pl.when(j + 3 < NCH)
        def _():
            _fetch_idx(j + 3)

        return carry

    lax.fori_loop(0, NCH, body, 0)
    _wait_scatter()
    plsc.subcore_barrier()
    pltpu.sync_copy(acc.at[pl.ds(s * RPT, RPT)], out.at[c, pl.ds(s * RPT, RPT)])


_sc_mesh = plsc.VectorSubcoreMesh(core_axis_name="c", subcore_axis_name="s")


def _deg(dstw):
    return pl.kernel(
        _deg_body,
        out_type=jax.ShapeDtypeStruct((NC, NPAD_D), jnp.float32),
        mesh=_sc_mesh,
        scratch_types=[
            pltpu.VMEM((NCH, CHUNK), jnp.int32),
            pltpu.VMEM((CHUNK,), jnp.float32),
            pltpu.VMEM((RPT_D,), jnp.float32),
            pltpu.VMEM_SHARED((NPAD_D,), jnp.float32),
            pltpu.SemaphoreType.DMA,
        ],
    )(dstw)


def _scatter(hs, ew):
    return pl.kernel(
        _scat_body,
        out_type=jax.ShapeDtypeStruct((NC, NPAD, H), jnp.float32),
        mesh=_sc_mesh,
        scratch_types=[
            pltpu.VMEM((3, CHUNK), jnp.int32),
            pltpu.VMEM((4, CHUNK), jnp.int32),
            pltpu.VMEM((3, CHUNK, H), jnp.float32),
            pltpu.VMEM_SHARED((NPAD, H), jnp.float32),
            pltpu.SemaphoreType.DMA,
            pltpu.SemaphoreType.DMA,
            pltpu.SemaphoreType.DMA,
        ],
    )(hs, ew)


def _tc1_body(x_ref, w1_ref, cnt_ref, hs_ref, dinv_ref):
    cnt = cnt_ref[0, :NPAD] + cnt_ref[1, :NPAD]
    dinv = lax.rsqrt(cnt + 1.0)[:, None]
    h = jnp.dot(x_ref[...], w1_ref[...], preferred_element_type=jnp.float32)
    hs_ref[...] = h * dinv
    dinv_ref[...] = dinv


def _tc2_body(a_ref, dinv_ref, b1_ref, w2_ref, hs2_ref):
    agg = a_ref[0] + a_ref[1]
    dinv = dinv_ref[...]
    h = jnp.maximum(agg * dinv + b1_ref[...], 0.0)
    hs2_ref[...] = jnp.dot(h, w2_ref[...], preferred_element_type=jnp.float32) * dinv


def _tc3_body(a_ref, dinv_ref, b2_ref, wl_ref, bl_ref, wq_ref, bq_ref,
              wg_ref, bg_ref, wp_ref, bp_ref, wt_ref, bt_ref,
              q_ref, g_ref, p_ref, t_ref):
    agg = a_ref[0] + a_ref[1]
    h2 = jnp.maximum(agg * dinv_ref[...] + b2_ref[...], 0.0)
    v = jnp.sum(h2[:N, :], axis=0, keepdims=True) * (1.0 / N)
    u = jnp.maximum(
        jnp.dot(v, wl_ref[...], preferred_element_type=jnp.float32)
        + bl_ref[...], 0.0)
    q_ref[...] = jnp.dot(u, wq_ref[...], preferred_element_type=jnp.float32) + bq_ref[...]
    g_ref[...] = jnp.dot(u, wg_ref[...], preferred_element_type=jnp.float32) + bg_ref[...]
    p_ref[...] = jnp.dot(u, wp_ref[...], preferred_element_type=jnp.float32) + bp_ref[...]
    t_ref[...] = jnp.dot(u, wt_ref[...], preferred_element_type=jnp.float32) + bt_ref[...]


def _f32(shape):
    return jax.ShapeDtypeStruct(shape, jnp.float32)


def kernel(x, edge_index, W1, b1, W2, b2, Wl, bl, Wq, bq, Wg, bg, Wp, bp, Wt, bt):
    src = edge_index[0].reshape(NW, EPW)
    dst = edge_index[1].reshape(NW, EPW)
    npads = NW * (SLOTS - EPW)
    pad_src = (jnp.arange(npads, dtype=jnp.int32) % N).reshape(NW, SLOTS - EPW)
    pad_dst = (N + (jnp.arange(npads, dtype=jnp.int32) % (NPAD - N))).reshape(
        NW, SLOTS - EPW)
    srcw = jnp.concatenate([src, pad_src], axis=1).reshape(NW, NCH, CHUNK)
    dstw = jnp.concatenate([dst, pad_dst], axis=1).reshape(NW, NCH, CHUNK)
    ew = jnp.stack([srcw, dstw], axis=1)
    x_p = jnp.pad(x, ((0, NPAD - N), (0, 0)))

    cnt = _deg(dstw)

    hs1, dinv = pl.pallas_call(
        _tc1_body,
        out_shape=(_f32((NPAD, H)), _f32((NPAD, 1))),
    )(x_p, W1, cnt)

    acc1 = _scatter(hs1, ew)

    hs2 = pl.pallas_call(
        _tc2_body,
        out_shape=_f32((NPAD, H)),
    )(acc1, dinv, b1.reshape(1, H), W2)

    acc2 = _scatter(hs2, ew)

    q, g, p, t = pl.pallas_call(
        _tc3_body,
        out_shape=(_f32((1, NQ)), _f32((1, NG)), _f32((1, NP)), _f32((1, NQ - 1))),
    )(acc2, dinv, b2.reshape(1, H), Wl, bl.reshape(1, H),
      Wq, bq.reshape(1, NQ), Wg, bg.reshape(1, NG),
      Wp, bp.reshape(1, NP), Wt, bt.reshape(1, NQ - 1))
    return (q.reshape(NQ), g.reshape(NG), p.reshape(NP), t.reshape(NQ - 1))


# speedup vs baseline: 1.4685x; 1.0013x over previous
import jax
import jax.numpy as jnp
from jax import lax
from jax.experimental import pallas as pl
from jax.experimental.pallas import tpu as pltpu
from jax.experimental.pallas import tpu_sc as plsc

N = 10000
E = 320000
D = 128
H = 128
NQ = 16
NG = 8
NP = 6

NC = 2
NS = 16
NW = NC * NS
EPW = E // NW
CHUNK = 128
NCH = 80
SLOTS = NCH * CHUNK
NPAD = 10112
RPT = NPAD // NS
NPAD_D = 10240
RPT_D = NPAD_D // NS


def _deg_body(dstw, out, dstv, onesv, zv, cnt, dsem):
    c = lax.axis_index("c")
    s = lax.axis_index("s")
    wid = s * NC + c
    pltpu.sync_copy(dstw.at[wid], dstv)
    for i in range(CHUNK // 16):
        onesv[pl.ds(i * 16, 16)] = jnp.ones((16,), jnp.float32)
    for i in range(RPT_D // 16):
        zv[pl.ds(i * 16, 16)] = jnp.zeros((16,), jnp.float32)
    pltpu.sync_copy(zv, cnt.at[pl.ds(s * RPT_D, RPT_D)])
    plsc.subcore_barrier()

    def body(j, carry):
        pltpu.async_copy(onesv, cnt.at[dstv.at[j]], dsem, add=True)

        @pl.when(j >= 8)
        def _():
            pltpu.make_async_copy(onesv, cnt.at[dstv.at[0]], dsem).wait()

        return carry

    lax.fori_loop(0, NCH, body, 0)
    for _ in range(8):
        pltpu.make_async_copy(onesv, cnt.at[dstv.at[0]], dsem).wait()
    plsc.subcore_barrier()
    pltpu.sync_copy(cnt.at[pl.ds(s * RPT_D, RPT_D)],
                    out.at[c, pl.ds(s * RPT_D, RPT_D)])


def _scat_body(hs, ew, out, srcv, dstv, rows, acc, isem, gsem, ssem):
    c = lax.axis_index("c")
    s = lax.axis_index("s")
    wid = s * NC + c
    def zrow(r, carry):
        for i in range(H // 16):
            rows[0, r, pl.ds(i * 16, 16)] = jnp.zeros((16,), jnp.float32)
        return carry

    lax.fori_loop(0, CHUNK, zrow, 0)
    base = s * RPT

    def _zfill(off, n):
        done = 0
        while done < n:
            step = min(CHUNK, n - done)
            pltpu.sync_copy(rows.at[0, pl.ds(0, step)],
                            acc.at[pl.ds(off + done, step)])
            done += step

    @pl.when(c == 0)
    def _():
        pltpu.sync_copy(hs.at[pl.ds(base, 320)], acc.at[pl.ds(base, 320)])
        _zfill(base + 320, 312)

    @pl.when(c != 0)
    def _():
        pltpu.sync_copy(hs.at[pl.ds(base + 320, 312)],
                        acc.at[pl.ds(base + 320, 312)])
        _zfill(base, 320)

    def _fetch_idx(j):
        pltpu.async_copy(ew.at[wid, 0, j], srcv.at[j % 3], isem)
        pltpu.async_copy(ew.at[wid, 1, j], dstv.at[j % 4], isem)

    _fetch_idx(0)
    _fetch_idx(1)
    _fetch_idx(2)

    def _wait_idx():
        pltpu.make_async_copy(ew.at[wid, 0, 0], srcv.at[0], isem).wait()
        pltpu.make_async_copy(ew.at[wid, 0, 0], srcv.at[0], isem).wait()

    def _wait_gather():
        pltpu.make_async_copy(hs.at[pl.ds(0, CHUNK)], rows.at[0], gsem).wait()

    def _wait_scatter():
        pltpu.make_async_copy(hs.at[pl.ds(0, CHUNK)], rows.at[0], ssem).wait()

    _wait_idx()
    pltpu.async_copy(hs.at[srcv.at[0]], rows.at[0], gsem)
    _wait_idx()
    pltpu.async_copy(hs.at[srcv.at[1]], rows.at[1], gsem)
    plsc.subcore_barrier()

    def body(j, carry):
        b = j % 3
        _wait_gather()

        @pl.when(j > 0)
        def _():
            _wait_scatter()

        pltpu.async_copy(rows.at[b], acc.at[dstv.at[j % 4]], ssem, add=True)

        @pl.when(j + 2 < NCH)
        def _():
            _wait_idx()
            pltpu.async_copy(hs.at[srcv.at[(j + 2) % 3]],
                             rows.at[(j + 2) % 3], gsem)

        @pl.when(j + 3 < NCH)
        def _():
            _fetch_idx(j + 3)

        return carry

    lax.fori_loop(0, NCH, body, 0)
    _wait_scatter()
    plsc.subcore_barrier()
    pltpu.sync_copy(acc.at[pl.ds(s * RPT, RPT)], out.at[c, pl.ds(s * RPT, RPT)])


_sc_mesh = plsc.VectorSubcoreMesh(core_axis_name="c", subcore_axis_name="s")


def _deg(dstw):
    return pl.kernel(
        _deg_body,
        out_type=jax.ShapeDtypeStruct((NC, NPAD_D), jnp.float32),
        mesh=_sc_mesh,
        scratch_types=[
            pltpu.VMEM((NCH, CHUNK), jnp.int32),
            pltpu.VMEM((CHUNK,), jnp.float32),
            pltpu.VMEM((RPT_D,), jnp.float32),
            pltpu.VMEM_SHARED((NPAD_D,), jnp.float32),
            pltpu.SemaphoreType.DMA,
        ],
    )(dstw)


def _scatter(hs, ew):
    return pl.kernel(
        _scat_body,
        out_type=jax.ShapeDtypeStruct((NC, NPAD, H), jnp.float32),
        mesh=_sc_mesh,
        scratch_types=[
            pltpu.VMEM((3, CHUNK), jnp.int32),
            pltpu.VMEM((4, CHUNK), jnp.int32),
            pltpu.VMEM((3, CHUNK, H), jnp.float32),
            pltpu.VMEM_SHARED((NPAD, H), jnp.float32),
            pltpu.SemaphoreType.DMA,
            pltpu.SemaphoreType.DMA,
            pltpu.SemaphoreType.DMA,
        ],
    )(hs, ew)


def _tc1_body(x_ref, w1_ref, cnt_ref, hs_ref, dinv_ref):
    cnt = cnt_ref[0, :NPAD] + cnt_ref[1, :NPAD]
    dinv = lax.rsqrt(cnt + 1.0)[:, None]
    h = jnp.dot(x_ref[...], w1_ref[...], preferred_element_type=jnp.float32)
    hs_ref[...] = h * dinv
    dinv_ref[...] = dinv


def _tc2_body(a_ref, dinv_ref, b1_ref, w2_ref, hs2_ref):
    agg = a_ref[0] + a_ref[1]
    dinv = dinv_ref[...]
    h = jnp.maximum(agg * dinv + b1_ref[...], 0.0)
    hs2_ref[...] = jnp.dot(h, w2_ref[...], preferred_element_type=jnp.float32) * dinv


def _tc3_body(a_ref, dinv_ref, b2_ref, wl_ref, bl_ref, wq_ref, bq_ref,
              wg_ref, bg_ref, wp_ref, bp_ref, wt_ref, bt_ref,
              q_ref, g_ref, p_ref, t_ref):
    agg = a_ref[0] + a_ref[1]
    h2 = jnp.maximum(agg * dinv_ref[...] + b2_ref[...], 0.0)
    v = jnp.sum(h2[:N, :], axis=0, keepdims=True) * (1.0 / N)
    u = jnp.maximum(
        jnp.dot(v, wl_ref[...], preferred_element_type=jnp.float32)
        + bl_ref[...], 0.0)
    q_ref[...] = jnp.dot(u, wq_ref[...], preferred_element_type=jnp.float32) + bq_ref[...]
    g_ref[...] = jnp.dot(u, wg_ref[...], preferred_element_type=jnp.float32) + bg_ref[...]
    p_ref[...] = jnp.dot(u, wp_ref[...], preferred_element_type=jnp.float32) + bp_ref[...]
    t_ref[...] = jnp.dot(u, wt_ref[...], preferred_element_type=jnp.float32) + bt_ref[...]


def _f32(shape):
    return jax.ShapeDtypeStruct(shape, jnp.float32)


def kernel(x, edge_index, W1, b1, W2, b2, Wl, bl, Wq, bq, Wg, bg, Wp, bp, Wt, bt):
    src = edge_index[0].reshape(NW, EPW)
    dst = edge_index[1].reshape(NW, EPW)
    npads = NW * (SLOTS - EPW)
    pad_src = (jnp.arange(npads, dtype=jnp.int32) % N).reshape(NW, SLOTS - EPW)
    pad_dst = (N + (jnp.arange(npads, dtype=jnp.int32) % (NPAD - N))).reshape(
        NW, SLOTS - EPW)
    srcw = jnp.concatenate([src, pad_src], axis=1).reshape(NW, NCH, CHUNK)
    dstw = jnp.concatenate([dst, pad_dst], axis=1).reshape(NW, NCH, CHUNK)
    ew = jnp.stack([srcw, dstw], axis=1)
    x_p = jnp.pad(x, ((0, NPAD - N), (0, 0)))

    cnt = _deg(dstw)

    hs1, dinv = pl.pallas_call(
        _tc1_body,
        out_shape=(_f32((NPAD, H)), _f32((NPAD, 1))),
    )(x_p, W1, cnt)

    acc1 = _scatter(hs1, ew)

    hs2 = pl.pallas_call(
        _tc2_body,
        out_shape=_f32((NPAD, H)),
    )(acc1, dinv, b1.reshape(1, H), W2)

    acc2 = _scatter(hs2, ew)

    q, g, p, t = pl.pallas_call(
        _tc3_body,
        out_shape=(_f32((1, NQ)), _f32((1, NG)), _f32((1, NP)), _f32((1, NQ - 1))),
    )(acc2, dinv, b2.reshape(1, H), Wl, bl.reshape(1, H),
      Wq, bq.reshape(1, NQ), Wg, bg.reshape(1, NG),
      Wp, bp.reshape(1, NP), Wt, bt.reshape(1, NQ - 1))
    return (q.reshape(NQ), g.reshape(NG), p.reshape(NP), t.reshape(NQ - 1))
